# baseline (speedup 1.0000x reference)
import functools

import jax
import jax.numpy as jnp
from jax import lax
from jax.experimental import pallas as pl
from jax.experimental.pallas import tpu as pltpu
from jax.experimental.pallas import tpu_sc as plsc

NUM_CODES = 5000
PAD_CODES = 5120
ROW_BLK = 1024
TAB_W = 128
N_CHUNKS = 4


def _prep_body(cbt_ref, wpo_ref, bpo_ref, wup_ref, bup_ref,
               a_ref, cbn_ref, lanef_ref, tab_ref):
    cbt = jnp.concatenate(
        [cbt_ref[...], jnp.zeros((8, PAD_CODES - NUM_CODES), jnp.float32)],
        axis=1)
    a_ref[...] = -2.0 * cbt
    cbn = jnp.sum(cbt * cbt, axis=0)
    lane = jax.lax.broadcasted_iota(jnp.int32, (1, PAD_CODES), 1)
    bias = jnp.where(lane >= NUM_CODES, jnp.float32(3e38), jnp.float32(0.0))
    cbn_ref[...] = cbn[None, :] + bias
    lanef_ref[...] = lane.astype(jnp.float32)
    wup = jnp.concatenate(
        [wup_ref[...], jnp.zeros((wup_ref.shape[0], TAB_W - wup_ref.shape[1]),
                                 jnp.float32)], axis=1)
    bup = jnp.concatenate(
        [bup_ref[...], jnp.zeros((1, TAB_W - bup_ref.shape[1]), jnp.float32)],
        axis=1)
    h2 = jax.lax.dot_general(cbt, wpo_ref[...],
                             (((0,), (0,)), ((), ())))
    h2 = h2 + bpo_ref[...]
    t = jnp.dot(h2, wup) + bup
    tab_ref[...] = jnp.clip(t, -1.0, 1.0)


def _make_argmin_body(base):
    def body(x_hbm, wd_ref, bd_ref, wpi_ref, bpi_ref, a_ref, cbn_ref,
             lanef_ref, idx_ref, xa_ref, xb_ref, sem):
        i = pl.program_id(0)
        n = pl.num_programs(0)
        bufs = [xa_ref, xb_ref]

        def blk_copy(step, buf):
            return pltpu.make_async_copy(
                x_hbm.at[pl.ds((base + step) * ROW_BLK, ROW_BLK), :],
                buf, sem)

        @pl.when(i == 0)
        def _():
            blk_copy(0, xa_ref).start()

        def compute(x_val):
            h = jnp.dot(x_val, wd_ref[...]) + bd_ref[...]
            z = jnp.dot(h, wpi_ref[...]) + bpi_ref[...]
            d = jnp.dot(z, a_ref[...]) + cbn_ref[...]
            lanef = lanef_ref[...]
            rmin = d[:, :128]
            ridx = jnp.broadcast_to(lanef[:, :128], rmin.shape)
            for j in range(1, PAD_CODES // 128):
                dj = d[:, j * 128:(j + 1) * 128]
                lj = jnp.broadcast_to(lanef[:, j * 128:(j + 1) * 128],
                                      dj.shape)
                ridx = jnp.where(dj < rmin, lj, ridx)
                rmin = jnp.minimum(dj, rmin)
            m = jnp.min(rmin, axis=1, keepdims=True)
            idxf = jnp.min(
                jnp.where(rmin == m, ridx, jnp.float32(65536.0)), axis=1)
            idx_ref[...] = idxf.astype(jnp.int32)

        par = lax.rem(i, 2)
        for p in (0, 1):
            @pl.when(par == p)
            def _(p=p):
                @pl.when(i + 1 < n)
                def _():
                    blk_copy(i + 1, bufs[1 - p]).start()
                blk_copy(i, bufs[p]).wait()

        compute(jnp.where(par == 0, xa_ref[...], xb_ref[...]))

    return body


def _sc_gather(table, idx2):
    n_rows, n_lanes = idx2.shape
    chunks_per_tile = n_rows // 32
    rows_per_tile = chunks_per_tile * n_lanes
    mesh = plsc.VectorSubcoreMesh(core_axis_name="c", subcore_axis_name="s")

    @functools.partial(
        pl.kernel, mesh=mesh,
        out_type=jax.ShapeDtypeStruct((n_rows * n_lanes, TAB_W), jnp.float32),
        scratch_types=[
            pltpu.VMEM((chunks_per_tile, n_lanes), jnp.int32),
            pltpu.VMEM((rows_per_tile, TAB_W), jnp.float32),
            pltpu.SemaphoreType.DMA,
        ])
    def gather_kernel(tab_hbm, idx_hbm, out_hbm, idx_v, rows_v, sem):
        wid = lax.axis_index("s") * 2 + lax.axis_index("c")
        pltpu.sync_copy(idx_hbm.at[pl.ds(wid * chunks_per_tile,
                                         chunks_per_tile)], idx_v)
        copies = [
            pltpu.async_copy(tab_hbm.at[idx_v.at[c]],
                             rows_v.at[pl.ds(c * n_lanes, n_lanes)], sem)
            for c in range(chunks_per_tile)
        ]
        for cp in copies:
            cp.wait()
        pltpu.sync_copy(rows_v,
                        out_hbm.at[pl.ds(wid * rows_per_tile, rows_per_tile)])

    return gather_kernel(table, idx2)


def kernel(x, W_down, b_down, W_pi, b_pi, codebook, W_po, b_po, W_up, b_up):
    B = x.shape[0]
    out_feat = W_up.shape[1]

    cbt_in = codebook.T
    a_op, cbn_row, lanef_row, table = pl.pallas_call(
        _prep_body,
        in_specs=[
            pl.BlockSpec(cbt_in.shape, lambda: (0, 0)),
            pl.BlockSpec(W_po.shape, lambda: (0, 0)),
            pl.BlockSpec((1, b_po.shape[0]), lambda: (0, 0)),
            pl.BlockSpec(W_up.shape, lambda: (0, 0)),
            pl.BlockSpec((1, b_up.shape[0]), lambda: (0, 0)),
        ],
        out_specs=[
            pl.BlockSpec((8, PAD_CODES), lambda: (0, 0)),
            pl.BlockSpec((1, PAD_CODES), lambda: (0, 0)),
            pl.BlockSpec((1, PAD_CODES), lambda: (0, 0)),
            pl.BlockSpec((PAD_CODES, TAB_W), lambda: (0, 0)),
        ],
        out_shape=[
            jax.ShapeDtypeStruct((8, PAD_CODES), jnp.float32),
            jax.ShapeDtypeStruct((1, PAD_CODES), jnp.float32),
            jax.ShapeDtypeStruct((1, PAD_CODES), jnp.float32),
            jax.ShapeDtypeStruct((PAD_CODES, TAB_W), jnp.float32),
        ],
    )(cbt_in, W_po, b_po.reshape(1, -1), W_up, b_up.reshape(1, -1))

    chunk = B // N_CHUNKS
    idx_parts, out_parts = [], []
    for c in range(N_CHUNKS):
        base = c * (chunk // ROW_BLK)
        idx_c = pl.pallas_call(
            _make_argmin_body(base),
            grid=(chunk // ROW_BLK,),
            in_specs=[
                pl.BlockSpec(memory_space=pl.ANY),
                pl.BlockSpec(W_down.shape, lambda i: (0, 0)),
                pl.BlockSpec((1, b_down.shape[0]), lambda i: (0, 0)),
                pl.BlockSpec(W_pi.shape, lambda i: (0, 0)),
                pl.BlockSpec((1, b_pi.shape[0]), lambda i: (0, 0)),
                pl.BlockSpec((8, PAD_CODES), lambda i: (0, 0)),
                pl.BlockSpec((1, PAD_CODES), lambda i: (0, 0)),
                pl.BlockSpec((1, PAD_CODES), lambda i: (0, 0)),
            ],
            out_specs=pl.BlockSpec((ROW_BLK,), lambda i: (i,)),
            out_shape=jax.ShapeDtypeStruct((chunk,), jnp.int32),
            scratch_shapes=[
                pltpu.VMEM((ROW_BLK, x.shape[1]), jnp.float32),
                pltpu.VMEM((ROW_BLK, x.shape[1]), jnp.float32),
                pltpu.SemaphoreType.DMA,
            ],
        )(x, W_down, b_down.reshape(1, -1), W_pi, b_pi.reshape(1, -1),
          a_op, cbn_row, lanef_row)
        idx_parts.append(idx_c)
        out_parts.append(_sc_gather(table, idx_c.reshape(chunk // 128, 128)))

    indices = jnp.concatenate(idx_parts)
    out = jnp.zeros((B, out_feat), jnp.float32)
    for c, p in enumerate(out_parts):
        out = lax.dynamic_update_slice(out, p[:, :out_feat], (c * chunk, 0))
    commit_loss = jnp.zeros((), dtype=jnp.float32)
    return (out, indices, commit_loss)


# speedup vs baseline: 1.1046x; 1.1046x over previous
import functools

import jax
import jax.numpy as jnp
from jax import lax
from jax.experimental import pallas as pl
from jax.experimental.pallas import tpu as pltpu
from jax.experimental.pallas import tpu_sc as plsc

NUM_CODES = 5000
PAD_CODES = 5120
ROW_BLK = 1024
TAB_W = 128
N_CHUNKS = 4


def _prep_body(cbt_ref, wpo_ref, bpo_ref, wup_ref, bup_ref,
               a_ref, cbn_ref, lanef_ref, tab_ref):
    cbt = jnp.concatenate(
        [cbt_ref[...], jnp.zeros((8, PAD_CODES - NUM_CODES), jnp.float32)],
        axis=1)
    a_ref[...] = -2.0 * cbt
    cbn = jnp.sum(cbt * cbt, axis=0)
    lane = jax.lax.broadcasted_iota(jnp.int32, (1, PAD_CODES), 1)
    bias = jnp.where(lane >= NUM_CODES, jnp.float32(3e38), jnp.float32(0.0))
    cbn_ref[...] = cbn[None, :] + bias
    lanef_ref[...] = lane.astype(jnp.float32)
    wup = jnp.concatenate(
        [wup_ref[...], jnp.zeros((wup_ref.shape[0], TAB_W - wup_ref.shape[1]),
                                 jnp.float32)], axis=1)
    bup = jnp.concatenate(
        [bup_ref[...], jnp.zeros((1, TAB_W - bup_ref.shape[1]), jnp.float32)],
        axis=1)
    h2 = jax.lax.dot_general(cbt, wpo_ref[...],
                             (((0,), (0,)), ((), ())))
    h2 = h2 + bpo_ref[...]
    t = jnp.dot(h2, wup) + bup
    tab_ref[...] = jnp.clip(t, -1.0, 1.0)


def _make_argmin_body():
    def body(xt_ref, wdt_ref, bd_ref, wpit_ref, bpi_ref, a_ref, cbn_ref,
             lanef_ref, idx_ref):
        def compute(xt_val):
            h = lax.dot_general(
                xt_val, wdt_ref[...],
                (((0,), (1,)), ((), ()))) + bd_ref[...]
            z = lax.dot_general(
                h, wpit_ref[...],
                (((1,), (1,)), ((), ()))) + bpi_ref[...]
            d = jnp.dot(z, a_ref[...]) + cbn_ref[...]
            lanef = lanef_ref[...]
            rmin = d[:, :128]
            ridx = jnp.broadcast_to(lanef[:, :128], rmin.shape)
            for j in range(1, PAD_CODES // 128):
                dj = d[:, j * 128:(j + 1) * 128]
                lj = jnp.broadcast_to(lanef[:, j * 128:(j + 1) * 128],
                                      dj.shape)
                ridx = jnp.where(dj < rmin, lj, ridx)
                rmin = jnp.minimum(dj, rmin)
            m = jnp.min(rmin, axis=1, keepdims=True)
            idxf = jnp.min(
                jnp.where(rmin == m, ridx, jnp.float32(65536.0)), axis=1)
            idx_ref[...] = idxf.astype(jnp.int32)

        compute(xt_ref[...])

    return body


def _sc_gather(table, idx2):
    n_rows, n_lanes = idx2.shape
    chunks_per_tile = n_rows // 32
    rows_per_tile = chunks_per_tile * n_lanes
    mesh = plsc.VectorSubcoreMesh(core_axis_name="c", subcore_axis_name="s")

    @functools.partial(
        pl.kernel, mesh=mesh,
        out_type=jax.ShapeDtypeStruct((n_rows * n_lanes, TAB_W), jnp.float32),
        scratch_types=[
            pltpu.VMEM((chunks_per_tile, n_lanes), jnp.int32),
            pltpu.VMEM((rows_per_tile, TAB_W), jnp.float32),
            pltpu.SemaphoreType.DMA,
        ])
    def gather_kernel(tab_hbm, idx_hbm, out_hbm, idx_v, rows_v, sem):
        wid = lax.axis_index("s") * 2 + lax.axis_index("c")
        pltpu.sync_copy(idx_hbm.at[pl.ds(wid * chunks_per_tile,
                                         chunks_per_tile)], idx_v)
        copies = [
            pltpu.async_copy(tab_hbm.at[idx_v.at[c]],
                             rows_v.at[pl.ds(c * n_lanes, n_lanes)], sem)
            for c in range(chunks_per_tile)
        ]
        for cp in copies:
            cp.wait()
        pltpu.sync_copy(rows_v,
                        out_hbm.at[pl.ds(wid * rows_per_tile, rows_per_tile)])

    return gather_kernel(table, idx2)


def kernel(x, W_down, b_down, W_pi, b_pi, codebook, W_po, b_po, W_up, b_up):
    B = x.shape[0]
    out_feat = W_up.shape[1]

    cbt_in = codebook.T
    a_op, cbn_row, lanef_row, table = pl.pallas_call(
        _prep_body,
        in_specs=[
            pl.BlockSpec(cbt_in.shape, lambda: (0, 0)),
            pl.BlockSpec(W_po.shape, lambda: (0, 0)),
            pl.BlockSpec((1, b_po.shape[0]), lambda: (0, 0)),
            pl.BlockSpec(W_up.shape, lambda: (0, 0)),
            pl.BlockSpec((1, b_up.shape[0]), lambda: (0, 0)),
        ],
        out_specs=[
            pl.BlockSpec((8, PAD_CODES), lambda: (0, 0)),
            pl.BlockSpec((1, PAD_CODES), lambda: (0, 0)),
            pl.BlockSpec((1, PAD_CODES), lambda: (0, 0)),
            pl.BlockSpec((PAD_CODES, TAB_W), lambda: (0, 0)),
        ],
        out_shape=[
            jax.ShapeDtypeStruct((8, PAD_CODES), jnp.float32),
            jax.ShapeDtypeStruct((1, PAD_CODES), jnp.float32),
            jax.ShapeDtypeStruct((1, PAD_CODES), jnp.float32),
            jax.ShapeDtypeStruct((PAD_CODES, TAB_W), jnp.float32),
        ],
    )(cbt_in, W_po, b_po.reshape(1, -1), W_up, b_up.reshape(1, -1))

    chunk = B // N_CHUNKS
    xt = x.T
    wdt = W_down.T
    wpit = W_pi.T
    idx_parts, out_parts = [], []
    for c in range(N_CHUNKS):
        base = c * (chunk // ROW_BLK)
        idx_c = pl.pallas_call(
            _make_argmin_body(),
            grid=(chunk // ROW_BLK,),
            in_specs=[
                pl.BlockSpec((x.shape[1], ROW_BLK),
                             lambda i, base=base: (0, i + base)),
                pl.BlockSpec(wdt.shape, lambda i: (0, 0)),
                pl.BlockSpec((1, b_down.shape[0]), lambda i: (0, 0)),
                pl.BlockSpec(wpit.shape, lambda i: (0, 0)),
                pl.BlockSpec((1, b_pi.shape[0]), lambda i: (0, 0)),
                pl.BlockSpec((8, PAD_CODES), lambda i: (0, 0)),
                pl.BlockSpec((1, PAD_CODES), lambda i: (0, 0)),
                pl.BlockSpec((1, PAD_CODES), lambda i: (0, 0)),
            ],
            out_specs=pl.BlockSpec((ROW_BLK,), lambda i: (i,)),
            out_shape=jax.ShapeDtypeStruct((chunk,), jnp.int32),
        )(xt, wdt, b_down.reshape(1, -1), wpit, b_pi.reshape(1, -1),
          a_op, cbn_row, lanef_row)
        idx_parts.append(idx_c)
        out_parts.append(_sc_gather(table, idx_c.reshape(chunk // 128, 128)))

    indices = jnp.concatenate(idx_parts)
    out = jnp.concatenate([p[:, :out_feat] for p in out_parts])
    commit_loss = jnp.zeros((), dtype=jnp.float32)
    return (out, indices, commit_loss)


# speedup vs baseline: 1.1744x; 1.0632x over previous
import functools

import jax
import jax.numpy as jnp
from jax import lax
from jax.experimental import pallas as pl
from jax.experimental.pallas import tpu as pltpu
from jax.experimental.pallas import tpu_sc as plsc

NUM_CODES = 5000
PAD_CODES = 5120
ROW_BLK = 1024
TAB_W = 128
N_CHUNKS = 4


def _prep_body(cbt_ref, wpo_ref, bpo_ref, wup_ref, bup_ref,
               a_ref, cbn_ref, lanef_ref, tab_ref):
    cbt = jnp.concatenate(
        [cbt_ref[...], jnp.zeros((8, PAD_CODES - NUM_CODES), jnp.float32)],
        axis=1)
    a_ref[...] = -2.0 * cbt
    cbn = jnp.sum(cbt * cbt, axis=0)
    lane = jax.lax.broadcasted_iota(jnp.int32, (1, PAD_CODES), 1)
    bias = jnp.where(lane >= NUM_CODES, jnp.float32(3e38), jnp.float32(0.0))
    cbn_ref[...] = cbn[None, :] + bias
    lanef_ref[...] = lane.astype(jnp.float32)
    wup = jnp.concatenate(
        [wup_ref[...], jnp.zeros((wup_ref.shape[0], TAB_W - wup_ref.shape[1]),
                                 jnp.float32)], axis=1)
    bup = jnp.concatenate(
        [bup_ref[...], jnp.zeros((1, TAB_W - bup_ref.shape[1]), jnp.float32)],
        axis=1)
    h2 = jax.lax.dot_general(cbt, wpo_ref[...],
                             (((0,), (0,)), ((), ())))
    h2 = h2 + bpo_ref[...]
    t = jnp.dot(h2, wup) + bup
    tab_ref[...] = jnp.clip(t, -1.0, 1.0)


def _make_argmin_body():
    def body(xt_ref, wdt_ref, bd_ref, wpit_ref, bpi_ref, a_ref, cbn_ref,
             lanef_ref, idx_ref):
        def compute(xt_val):
            h = lax.dot_general(
                xt_val, wdt_ref[...],
                (((0,), (1,)), ((), ()))) + bd_ref[...]
            z = lax.dot_general(
                h, wpit_ref[...],
                (((1,), (1,)), ((), ()))) + bpi_ref[...]
            d = jnp.dot(z, a_ref[...]) + cbn_ref[...]
            lanef = lanef_ref[...]
            rmin = d[:, :128]
            ridx = jnp.broadcast_to(lanef[:, :128], rmin.shape)
            for j in range(1, PAD_CODES // 128):
                dj = d[:, j * 128:(j + 1) * 128]
                lj = jnp.broadcast_to(lanef[:, j * 128:(j + 1) * 128],
                                      dj.shape)
                ridx = jnp.where(dj < rmin, lj, ridx)
                rmin = jnp.minimum(dj, rmin)
            m = jnp.min(rmin, axis=1, keepdims=True)
            idxf = jnp.min(
                jnp.where(rmin == m, ridx, jnp.float32(65536.0)), axis=1)
            idx_ref[...] = idxf.astype(jnp.int32)

        compute(xt_ref[...])

    return body


def _trans_body(*refs):
    o_ref = refs[-1]
    parts = refs[:-1]
    n = parts[0].shape[0]
    for c, p in enumerate(parts):
        o_ref[:, c * n:(c + 1) * n] = p[...][:, :o_ref.shape[0]].T


def _sc_gather(table, idx2):
    n_rows, n_lanes = idx2.shape
    chunks_per_tile = n_rows // 32
    rows_per_tile = chunks_per_tile * n_lanes
    mesh = plsc.VectorSubcoreMesh(core_axis_name="c", subcore_axis_name="s")

    @functools.partial(
        pl.kernel, mesh=mesh,
        out_type=jax.ShapeDtypeStruct((n_rows * n_lanes, TAB_W), jnp.float32),
        scratch_types=[
            pltpu.VMEM((chunks_per_tile, n_lanes), jnp.int32),
            pltpu.VMEM((rows_per_tile, TAB_W), jnp.float32),
            pltpu.SemaphoreType.DMA,
        ])
    def gather_kernel(tab_hbm, idx_hbm, out_hbm, idx_v, rows_v, sem):
        wid = lax.axis_index("s") * 2 + lax.axis_index("c")
        pltpu.sync_copy(idx_hbm.at[pl.ds(wid * chunks_per_tile,
                                         chunks_per_tile)], idx_v)
        copies = [
            pltpu.async_copy(tab_hbm.at[idx_v.at[c]],
                             rows_v.at[pl.ds(c * n_lanes, n_lanes)], sem)
            for c in range(chunks_per_tile)
        ]
        for cp in copies:
            cp.wait()
        pltpu.sync_copy(rows_v,
                        out_hbm.at[pl.ds(wid * rows_per_tile, rows_per_tile)])

    return gather_kernel(table, idx2)


def kernel(x, W_down, b_down, W_pi, b_pi, codebook, W_po, b_po, W_up, b_up):
    B = x.shape[0]
    out_feat = W_up.shape[1]

    cbt_in = codebook.T
    a_op, cbn_row, lanef_row, table = pl.pallas_call(
        _prep_body,
        in_specs=[
            pl.BlockSpec(cbt_in.shape, lambda: (0, 0)),
            pl.BlockSpec(W_po.shape, lambda: (0, 0)),
            pl.BlockSpec((1, b_po.shape[0]), lambda: (0, 0)),
            pl.BlockSpec(W_up.shape, lambda: (0, 0)),
            pl.BlockSpec((1, b_up.shape[0]), lambda: (0, 0)),
        ],
        out_specs=[
            pl.BlockSpec((8, PAD_CODES), lambda: (0, 0)),
            pl.BlockSpec((1, PAD_CODES), lambda: (0, 0)),
            pl.BlockSpec((1, PAD_CODES), lambda: (0, 0)),
            pl.BlockSpec((PAD_CODES, TAB_W), lambda: (0, 0)),
        ],
        out_shape=[
            jax.ShapeDtypeStruct((8, PAD_CODES), jnp.float32),
            jax.ShapeDtypeStruct((1, PAD_CODES), jnp.float32),
            jax.ShapeDtypeStruct((1, PAD_CODES), jnp.float32),
            jax.ShapeDtypeStruct((PAD_CODES, TAB_W), jnp.float32),
        ],
    )(cbt_in, W_po, b_po.reshape(1, -1), W_up, b_up.reshape(1, -1))

    chunk = B // N_CHUNKS
    xt = x.T
    wdt = W_down.T
    wpit = W_pi.T
    idx_parts, out_parts = [], []
    for c in range(N_CHUNKS):
        base = c * (chunk // ROW_BLK)
        idx_c = pl.pallas_call(
            _make_argmin_body(),
            grid=(chunk // ROW_BLK,),
            in_specs=[
                pl.BlockSpec((x.shape[1], ROW_BLK),
                             lambda i, base=base: (0, i + base)),
                pl.BlockSpec(wdt.shape, lambda i: (0, 0)),
                pl.BlockSpec((1, b_down.shape[0]), lambda i: (0, 0)),
                pl.BlockSpec(wpit.shape, lambda i: (0, 0)),
                pl.BlockSpec((1, b_pi.shape[0]), lambda i: (0, 0)),
                pl.BlockSpec((8, PAD_CODES), lambda i: (0, 0)),
                pl.BlockSpec((1, PAD_CODES), lambda i: (0, 0)),
                pl.BlockSpec((1, PAD_CODES), lambda i: (0, 0)),
            ],
            out_specs=pl.BlockSpec((ROW_BLK,), lambda i: (i,)),
            out_shape=jax.ShapeDtypeStruct((chunk,), jnp.int32),
        )(xt, wdt, b_down.reshape(1, -1), wpit, b_pi.reshape(1, -1),
          a_op, cbn_row, lanef_row)
        idx_parts.append(idx_c)
        out_parts.append(_sc_gather(table, idx_c.reshape(chunk // 128, 128)))

    indices = jnp.concatenate(idx_parts)
    out_t = pl.pallas_call(
        _trans_body,
        in_specs=[pl.BlockSpec((chunk, TAB_W), lambda: (0, 0))
                  for _ in range(N_CHUNKS)],
        out_specs=pl.BlockSpec((out_feat, B), lambda: (0, 0)),
        out_shape=jax.ShapeDtypeStruct((out_feat, B), jnp.float32),
    )(*out_parts)
    out = out_t.T
    commit_loss = jnp.zeros((), dtype=jnp.float32)
    return (out, indices, commit_loss)


# speedup vs baseline: 1.1847x; 1.0088x over previous
import functools

import jax
import jax.numpy as jnp
from jax import lax
from jax.experimental import pallas as pl
from jax.experimental.pallas import tpu as pltpu
from jax.experimental.pallas import tpu_sc as plsc

NUM_CODES = 5000
PAD_CODES = 5120
ROW_BLK = 1024
TAB_W = 128
N_CHUNKS = 8


def _prep_body(cbt_ref, wpo_ref, bpo_ref, wup_ref, bup_ref,
               a_ref, cbn_ref, lanef_ref, tab_ref):
    cbt = jnp.concatenate(
        [cbt_ref[...], jnp.zeros((8, PAD_CODES - NUM_CODES), jnp.float32)],
        axis=1)
    a_ref[...] = -2.0 * cbt
    cbn = jnp.sum(cbt * cbt, axis=0)
    lane = jax.lax.broadcasted_iota(jnp.int32, (1, PAD_CODES), 1)
    bias = jnp.where(lane >= NUM_CODES, jnp.float32(3e38), jnp.float32(0.0))
    cbn_ref[...] = cbn[None, :] + bias
    lanef_ref[...] = lane.astype(jnp.float32)
    wup = jnp.concatenate(
        [wup_ref[...], jnp.zeros((wup_ref.shape[0], TAB_W - wup_ref.shape[1]),
                                 jnp.float32)], axis=1)
    bup = jnp.concatenate(
        [bup_ref[...], jnp.zeros((1, TAB_W - bup_ref.shape[1]), jnp.float32)],
        axis=1)
    h2 = jax.lax.dot_general(cbt, wpo_ref[...],
                             (((0,), (0,)), ((), ())))
    h2 = h2 + bpo_ref[...]
    t = jnp.dot(h2, wup) + bup
    tab_ref[...] = jnp.clip(t, -1.0, 1.0)


def _make_argmin_body():
    def body(xt_ref, wdt_ref, bd_ref, wpit_ref, bpi_ref, a_ref, cbn_ref,
             lanef_ref, idx_ref):
        def compute(xt_val):
            h = lax.dot_general(
                xt_val, wdt_ref[...],
                (((0,), (1,)), ((), ()))) + bd_ref[...]
            z = lax.dot_general(
                h, wpit_ref[...],
                (((1,), (1,)), ((), ()))) + bpi_ref[...]
            d = jnp.dot(z, a_ref[...]) + cbn_ref[...]
            lanef = lanef_ref[...]
            rmin = d[:, :128]
            ridx = jnp.broadcast_to(lanef[:, :128], rmin.shape)
            for j in range(1, PAD_CODES // 128):
                dj = d[:, j * 128:(j + 1) * 128]
                lj = jnp.broadcast_to(lanef[:, j * 128:(j + 1) * 128],
                                      dj.shape)
                ridx = jnp.where(dj < rmin, lj, ridx)
                rmin = jnp.minimum(dj, rmin)
            m = jnp.min(rmin, axis=1, keepdims=True)
            idxf = jnp.min(
                jnp.where(rmin == m, ridx, jnp.float32(65536.0)), axis=1)
            idx_ref[...] = idxf.astype(jnp.int32)

        compute(xt_ref[...])

    return body


def _trans_body(*refs):
    o_ref = refs[-1]
    parts = refs[:-1]
    n = parts[0].shape[0]
    for c, p in enumerate(parts):
        o_ref[:, c * n:(c + 1) * n] = p[:, :o_ref.shape[0]][...].T


def _sc_gather(table, idx2):
    n_rows, n_lanes = idx2.shape
    chunks_per_tile = n_rows // 32
    rows_per_tile = chunks_per_tile * n_lanes
    mesh = plsc.VectorSubcoreMesh(core_axis_name="c", subcore_axis_name="s")

    @functools.partial(
        pl.kernel, mesh=mesh,
        out_type=jax.ShapeDtypeStruct((n_rows * n_lanes, TAB_W), jnp.float32),
        scratch_types=[
            pltpu.VMEM((chunks_per_tile, n_lanes), jnp.int32),
            pltpu.VMEM((rows_per_tile, TAB_W), jnp.float32),
            pltpu.SemaphoreType.DMA,
        ])
    def gather_kernel(tab_hbm, idx_hbm, out_hbm, idx_v, rows_v, sem):
        wid = lax.axis_index("s") * 2 + lax.axis_index("c")
        pltpu.sync_copy(idx_hbm.at[pl.ds(wid * chunks_per_tile,
                                         chunks_per_tile)], idx_v)
        copies = [
            pltpu.async_copy(tab_hbm.at[idx_v.at[c]],
                             rows_v.at[pl.ds(c * n_lanes, n_lanes)], sem)
            for c in range(chunks_per_tile)
        ]
        for cp in copies:
            cp.wait()
        pltpu.sync_copy(rows_v,
                        out_hbm.at[pl.ds(wid * rows_per_tile, rows_per_tile)])

    return gather_kernel(table, idx2)


def kernel(x, W_down, b_down, W_pi, b_pi, codebook, W_po, b_po, W_up, b_up):
    B = x.shape[0]
    out_feat = W_up.shape[1]

    cbt_in = codebook.T
    a_op, cbn_row, lanef_row, table = pl.pallas_call(
        _prep_body,
        in_specs=[
            pl.BlockSpec(cbt_in.shape, lambda: (0, 0)),
            pl.BlockSpec(W_po.shape, lambda: (0, 0)),
            pl.BlockSpec((1, b_po.shape[0]), lambda: (0, 0)),
            pl.BlockSpec(W_up.shape, lambda: (0, 0)),
            pl.BlockSpec((1, b_up.shape[0]), lambda: (0, 0)),
        ],
        out_specs=[
            pl.BlockSpec((8, PAD_CODES), lambda: (0, 0)),
            pl.BlockSpec((1, PAD_CODES), lambda: (0, 0)),
            pl.BlockSpec((1, PAD_CODES), lambda: (0, 0)),
            pl.BlockSpec((PAD_CODES, TAB_W), lambda: (0, 0)),
        ],
        out_shape=[
            jax.ShapeDtypeStruct((8, PAD_CODES), jnp.float32),
            jax.ShapeDtypeStruct((1, PAD_CODES), jnp.float32),
            jax.ShapeDtypeStruct((1, PAD_CODES), jnp.float32),
            jax.ShapeDtypeStruct((PAD_CODES, TAB_W), jnp.float32),
        ],
    )(cbt_in, W_po, b_po.reshape(1, -1), W_up, b_up.reshape(1, -1))

    chunk = B // N_CHUNKS
    xt = x.T
    wdt = W_down.T
    wpit = W_pi.T
    idx_parts, out_parts = [], []
    for c in range(N_CHUNKS):
        base = c * (chunk // ROW_BLK)
        idx_c = pl.pallas_call(
            _make_argmin_body(),
            grid=(chunk // ROW_BLK,),
            in_specs=[
                pl.BlockSpec((x.shape[1], ROW_BLK),
                             lambda i, base=base: (0, i + base)),
                pl.BlockSpec(wdt.shape, lambda i: (0, 0)),
                pl.BlockSpec((1, b_down.shape[0]), lambda i: (0, 0)),
                pl.BlockSpec(wpit.shape, lambda i: (0, 0)),
                pl.BlockSpec((1, b_pi.shape[0]), lambda i: (0, 0)),
                pl.BlockSpec((8, PAD_CODES), lambda i: (0, 0)),
                pl.BlockSpec((1, PAD_CODES), lambda i: (0, 0)),
                pl.BlockSpec((1, PAD_CODES), lambda i: (0, 0)),
            ],
            out_specs=pl.BlockSpec((ROW_BLK,), lambda i: (i,)),
            out_shape=jax.ShapeDtypeStruct((chunk,), jnp.int32),
        )(xt, wdt, b_down.reshape(1, -1), wpit, b_pi.reshape(1, -1),
          a_op, cbn_row, lanef_row)
        idx_parts.append(idx_c)
        out_parts.append(_sc_gather(table, idx_c.reshape(chunk // 128, 128)))

    indices = jnp.concatenate(idx_parts)
    out_t = pl.pallas_call(
        _trans_body,
        in_specs=[pl.BlockSpec((chunk, TAB_W), lambda: (0, 0))
                  for _ in range(N_CHUNKS)],
        out_specs=pl.BlockSpec((out_feat, B), lambda: (0, 0)),
        out_shape=jax.ShapeDtypeStruct((out_feat, B), jnp.float32),
    )(*out_parts)
    out = out_t.T
    commit_loss = jnp.zeros((), dtype=jnp.float32)
    return (out, indices, commit_loss)
